# baseline (speedup 1.0000x reference)
import jax
import jax.numpy as jnp
from jax.experimental import pallas as pl

DIM = 128
NUM_Q = 8
CB = 1024
TBLK = 512


def _rvq_kernel(x_ref, cb_ref, q_ref, codes_ref, loss_ref):
    x = x_ref[...]
    r = x
    qsum = jnp.zeros_like(x)
    loss = jnp.zeros((8, 128), dtype=jnp.float32)
    iota = jax.lax.broadcasted_iota(jnp.int32, (TBLK, CB), 1)
    for q in range(NUM_Q):
        cb = cb_ref[q]
        cn = jnp.sum(cb * cb, axis=1)
        rn = jnp.sum(r * r, axis=1, keepdims=True)
        s = jax.lax.dot_general(
            r, cb, (((1,), (1,)), ((), ())),
            precision=jax.lax.Precision.DEFAULT,
            preferred_element_type=jnp.float32,
        )
        dist = (rn - 2.0 * s) + cn[None, :]
        m = jnp.min(dist, axis=1, keepdims=True)
        idx = jnp.min(jnp.where(dist == m, iota, CB), axis=1, keepdims=True)
        onehot = (iota == idx).astype(jnp.float32)
        quant = jax.lax.dot_general(
            onehot, cb, (((1,), (0,)), ((), ())),
            precision=jax.lax.Precision.HIGHEST,
            preferred_element_type=jnp.float32,
        )
        codes_ref[:, q:q + 1] = idx
        r = r - quant
        qsum = qsum + quant
        rr = r * r
        loss = loss + jnp.sum(rr.reshape(TBLK // 8, 8, DIM), axis=0)

    q_ref[...] = x + (qsum - x)

    @pl.when(pl.program_id(0) == 0)
    def _init():
        loss_ref[...] = jnp.zeros_like(loss_ref)

    loss_ref[...] += loss


def kernel(latents, codebooks):
    B, C, T = latents.shape
    N = B * T
    flat = latents.transpose(0, 2, 1).reshape(N, C)
    grid = (N // TBLK,)
    q_flat, codes_flat, loss_sum = pl.pallas_call(
        _rvq_kernel,
        grid=grid,
        in_specs=[
            pl.BlockSpec((TBLK, DIM), lambda i: (i, 0)),
            pl.BlockSpec((NUM_Q, CB, DIM), lambda i: (0, 0, 0)),
        ],
        out_specs=[
            pl.BlockSpec((TBLK, DIM), lambda i: (i, 0)),
            pl.BlockSpec((TBLK, NUM_Q), lambda i: (i, 0)),
            pl.BlockSpec((8, 128), lambda i: (0, 0)),
        ],
        out_shape=[
            jax.ShapeDtypeStruct((N, DIM), jnp.float32),
            jax.ShapeDtypeStruct((N, NUM_Q), jnp.int32),
            jax.ShapeDtypeStruct((8, 128), jnp.float32),
        ],
    )(flat, codebooks)
    quantized = q_flat.reshape(B, T, C).transpose(0, 2, 1)
    codes = codes_flat.reshape(B, T, NUM_Q).transpose(0, 2, 1)
    denom = jnp.float32(N * C * NUM_Q)
    loss = jnp.sum(loss_sum) / denom
    return quantized, codes, loss, loss


# speedup vs baseline: 1.7928x; 1.7928x over previous
import jax
import jax.numpy as jnp
from jax.experimental import pallas as pl

DIM = 128
NUM_Q = 8
CB = 1024
TBLK = 1024


def _rvq_kernel(x_ref, cb_ref, cbh_ref, cbm_ref, cbl_ref, cn_ref, q_ref,
                codes_ref, loss_ref):
    x = x_ref[...]
    r = x
    qsum = jnp.zeros_like(x)
    loss = jnp.zeros((8, 128), dtype=jnp.float32)
    iota = jax.lax.broadcasted_iota(jnp.int32, (TBLK, CB), 1).astype(jnp.float32)
    for q in range(NUM_Q):
        cb = cb_ref[q]
        cn = cn_ref[q]
        rn = jnp.sum(r * r, axis=1, keepdims=True)
        s = jax.lax.dot_general(
            r, cb, (((1,), (1,)), ((), ())),
            precision=jax.lax.Precision.DEFAULT,
            preferred_element_type=jnp.float32,
        )
        dist = (rn - 2.0 * s) + cn[None, :]
        m = jnp.min(dist, axis=1, keepdims=True)
        idx = jnp.min(jnp.where(dist == m, iota, jnp.float32(CB)), axis=1,
                      keepdims=True)
        onehot = (iota == idx).astype(jnp.bfloat16)
        gathered = [
            jax.lax.dot_general(
                onehot, part[q], (((1,), (0,)), ((), ())),
                preferred_element_type=jnp.float32)
            for part in (cbh_ref, cbm_ref, cbl_ref)
        ]
        quant = (gathered[0] + gathered[1]) + gathered[2]
        codes_ref[:, q:q + 1] = idx.astype(jnp.int32)
        r = r - quant
        qsum = qsum + quant
        rr = r * r
        loss = loss + jnp.sum(rr.reshape(TBLK // 8, 8, DIM), axis=0)

    q_ref[...] = x + (qsum - x)

    @pl.when(pl.program_id(0) == 0)
    def _init():
        loss_ref[...] = jnp.zeros_like(loss_ref)

    loss_ref[...] += loss


def kernel(latents, codebooks):
    B, C, T = latents.shape
    N = B * T
    flat = latents.transpose(0, 2, 1).reshape(N, C)

    def _trunc16(v):
        u = jax.lax.bitcast_convert_type(v, jnp.uint32)
        return jax.lax.bitcast_convert_type(u & jnp.uint32(0xFFFF0000),
                                            jnp.float32)

    c1 = _trunc16(codebooks)
    r1 = codebooks - c1
    c2 = _trunc16(r1)
    c3 = r1 - c2
    cb_hi, cb_mid, cb_lo = (c.astype(jnp.bfloat16) for c in (c1, c2, c3))
    cn = jnp.sum(codebooks * codebooks, axis=-1)
    grid = (N // TBLK,)
    q_flat, codes_flat, loss_sum = pl.pallas_call(
        _rvq_kernel,
        grid=grid,
        in_specs=[
            pl.BlockSpec((TBLK, DIM), lambda i: (i, 0)),
            pl.BlockSpec((NUM_Q, CB, DIM), lambda i: (0, 0, 0)),
            pl.BlockSpec((NUM_Q, CB, DIM), lambda i: (0, 0, 0)),
            pl.BlockSpec((NUM_Q, CB, DIM), lambda i: (0, 0, 0)),
            pl.BlockSpec((NUM_Q, CB, DIM), lambda i: (0, 0, 0)),
            pl.BlockSpec((NUM_Q, CB), lambda i: (0, 0)),
        ],
        out_specs=[
            pl.BlockSpec((TBLK, DIM), lambda i: (i, 0)),
            pl.BlockSpec((TBLK, NUM_Q), lambda i: (i, 0)),
            pl.BlockSpec((8, 128), lambda i: (0, 0)),
        ],
        out_shape=[
            jax.ShapeDtypeStruct((N, DIM), jnp.float32),
            jax.ShapeDtypeStruct((N, NUM_Q), jnp.int32),
            jax.ShapeDtypeStruct((8, 128), jnp.float32),
        ],
    )(flat, codebooks, cb_hi, cb_mid, cb_lo, cn)
    quantized = q_flat.reshape(B, T, C).transpose(0, 2, 1)
    codes = codes_flat.reshape(B, T, NUM_Q).transpose(0, 2, 1)
    denom = jnp.float32(N * C * NUM_Q)
    loss = jnp.sum(loss_sum) / denom
    return quantized, codes, loss, loss


# speedup vs baseline: 3.5824x; 1.9982x over previous
import jax
import jax.numpy as jnp
from jax.experimental import pallas as pl

DIM = 128
NUM_Q = 8
CB = 1024
TBLK = 1024
NH = 2
H = TBLK // NH


def _rvq_kernel(x_ref, cb_ref, cbcat_ref, cn_ref, q_ref, codes_ref, loss_ref):
    x = x_ref[...]
    rs = [x[h * H:(h + 1) * H] for h in range(NH)]
    qsums = [jnp.zeros((H, DIM), jnp.float32) for _ in range(NH)]
    loss = jnp.zeros((8, 128), dtype=jnp.float32)
    iota = jax.lax.broadcasted_iota(jnp.int32, (H, CB), 1).astype(jnp.float32)
    for q in range(NUM_Q):
        cb = cb_ref[q]
        cbcat = cbcat_ref[q]
        cn = cn_ref[q]
        ss = [
            jax.lax.dot_general(
                r, cb, (((1,), (1,)), ((), ())),
                precision=jax.lax.Precision.DEFAULT,
                preferred_element_type=jnp.float32,
            ) for r in rs
        ]
        rns = [jnp.sum(r * r, axis=1, keepdims=True) for r in rs]
        dists = [(rn - 2.0 * s) + cn[None, :] for rn, s in zip(rns, ss)]
        ms = [jnp.min(d, axis=1, keepdims=True) for d in dists]
        maskeds = [
            jnp.where(d == m, iota, jnp.float32(CB))
            for d, m in zip(dists, ms)
        ]
        idxs = [jnp.min(mk, axis=1, keepdims=True) for mk in maskeds]
        onehots = [
            (mk == ix).astype(jnp.float32) for mk, ix in zip(maskeds, idxs)
        ]
        gs = [
            jax.lax.dot_general(
                oh, cbcat, (((1,), (0,)), ((), ())),
                precision=jax.lax.Precision.DEFAULT,
                preferred_element_type=jnp.float32,
            ) for oh in onehots
        ]
        for h in range(NH):
            g = gs[h]
            quant = (g[:, :DIM] + g[:, DIM:2 * DIM]) + g[:, 2 * DIM:]
            codes_ref[h * H:(h + 1) * H, q:q + 1] = idxs[h].astype(jnp.int32)
            rs[h] = rs[h] - quant
            qsums[h] = qsums[h] + quant
            rr = rs[h] * rs[h]
            loss = loss + jnp.sum(rr.reshape(H // 8, 8, DIM), axis=0)

    for h in range(NH):
        xh = x[h * H:(h + 1) * H]
        q_ref[h * H:(h + 1) * H, :] = xh + (qsums[h] - xh)

    @pl.when(pl.program_id(0) == 0)
    def _init():
        loss_ref[...] = jnp.zeros_like(loss_ref)

    loss_ref[...] += loss


def kernel(latents, codebooks):
    B, C, T = latents.shape
    N = B * T
    flat = latents.transpose(0, 2, 1).reshape(N, C)

    def _trunc16(v):
        u = jax.lax.bitcast_convert_type(v, jnp.uint32)
        return jax.lax.bitcast_convert_type(u & jnp.uint32(0xFFFF0000),
                                            jnp.float32)

    c1 = _trunc16(codebooks)
    r1 = codebooks - c1
    c2 = _trunc16(r1)
    c3 = r1 - c2
    cbcat = jnp.concatenate([c1, c2, c3], axis=-1)
    cn = jnp.sum(codebooks * codebooks, axis=-1)
    grid = (N // TBLK,)
    q_flat, codes_flat, loss_sum = pl.pallas_call(
        _rvq_kernel,
        grid=grid,
        in_specs=[
            pl.BlockSpec((TBLK, DIM), lambda i: (i, 0)),
            pl.BlockSpec((NUM_Q, CB, DIM), lambda i: (0, 0, 0)),
            pl.BlockSpec((NUM_Q, CB, 3 * DIM), lambda i: (0, 0, 0)),
            pl.BlockSpec((NUM_Q, CB), lambda i: (0, 0)),
        ],
        out_specs=[
            pl.BlockSpec((TBLK, DIM), lambda i: (i, 0)),
            pl.BlockSpec((TBLK, NUM_Q), lambda i: (i, 0)),
            pl.BlockSpec((8, 128), lambda i: (0, 0)),
        ],
        out_shape=[
            jax.ShapeDtypeStruct((N, DIM), jnp.float32),
            jax.ShapeDtypeStruct((N, NUM_Q), jnp.int32),
            jax.ShapeDtypeStruct((8, 128), jnp.float32),
        ],
    )(flat, codebooks, cbcat, cn)
    quantized = q_flat.reshape(B, T, C).transpose(0, 2, 1)
    codes = codes_flat.reshape(B, T, NUM_Q).transpose(0, 2, 1)
    denom = jnp.float32(N * C * NUM_Q)
    loss = jnp.sum(loss_sum) / denom
    return quantized, codes, loss, loss
